# Initial kernel scaffold; baseline (speedup 1.0000x reference)
import jax
import jax.numpy as jnp
from jax.experimental import pallas as pl


def kernel(node_embeddings, time_positions, node_pred, audio_len, cur_anchor_intervals, params, num_intervals_per_scale):
    raise NotImplementedError("write your pallas kernel here")


# speedup vs baseline: 22.6385x; 22.6385x over previous
import jax
import jax.numpy as jnp
import numpy as np
from jax.experimental import pallas as pl
from jax.experimental.pallas import tpu as pltpu

_SMOOTH = np.array([0.06136, 0.24477, 0.38774, 0.24477, 0.06136], dtype=np.float32)
_NIP = (30, 24, 16)
_NPAD = (32, 24, 16)
_ROW0 = (0, 32, 56)
_NBINS = (80, 60, 40)
_NT = 4096
_D = 128


def _body(*refs):
    it = iter(refs)
    ne_ref = next(it)
    tp_ref = next(it)
    tprow_ref = next(it)
    au_ref = next(it)
    scol_ref = next(it)
    ecol_ref = next(it)
    npp_ref = next(it)
    wihT = [next(it) for _ in range(3)]
    whhT = [next(it) for _ in range(3)]
    gb = [next(it) for _ in range(3)]
    bhn = [next(it) for _ in range(3)]
    giaw_ref = next(it)
    giab_ref = next(it)
    abhh_ref = next(it)
    refw = [[(next(it), next(it), next(it), next(it)) for _ in range(3)]
            for _ in range(3)]
    wp = [next(it) for _ in range(3)]
    heads = [(next(it), next(it), next(it), next(it)) for _ in range(3)]
    out_ref = next(it)
    gi_ref = [next(it) for _ in range(3)]
    gia_ref = next(it)

    al = au_ref[:]
    s_col = scol_ref[:]
    e_col = ecol_ref[:]

    x = npp_ref[:]
    zpad = jnp.zeros((2, 5), jnp.float32)
    xp = jnp.concatenate([zpad, x, zpad], axis=0)
    sm = (_SMOOTH[0] * xp[0:_NT] + _SMOOTH[1] * xp[1:_NT + 1]
          + _SMOOTH[2] * xp[2:_NT + 2] + _SMOOTH[3] * xp[3:_NT + 3]
          + _SMOOTH[4] * xp[4:_NT + 4])
    smax = jax.nn.softmax(sm, axis=1)
    ab_col = 1.0 - smax[:, 0:1]
    gia_ref[:] = ab_col * giaw_ref[:] + giab_ref[:]

    nev = ne_ref[:]
    for s in range(3):
        gi_ref[s][:] = jnp.dot(nev, wihT[s][:],
                               preferred_element_type=jnp.float32) + gb[s][:]

    t_row = tprow_ref[:] * al
    inmask = (t_row >= s_col) & (t_row <= e_col)
    counts = jnp.sum(inmask.astype(jnp.float32), axis=1, keepdims=True)

    h_emp = []
    for s in range(3):
        gbv = gb[s][:]
        r0 = jax.nn.sigmoid(gbv[:, 0:128])
        z0 = jax.nn.sigmoid(gbv[:, 128:256])
        n0 = jnp.tanh(gbv[:, 256:384] + r0 * bhn[s][:])
        h_emp.append((1.0 - z0) * n0)
    abhh = abhh_ref[:]
    giab = giab_ref[:]
    a_emp = []
    for s in range(3):
        r0 = jax.nn.sigmoid(giab[:, 3 * s:3 * s + 1])
        z0 = jax.nn.sigmoid(giab[:, 3 * s + 1:3 * s + 2])
        n0 = jnp.tanh(giab[:, 3 * s + 2:3 * s + 3]
                      + r0 * abhh[:, 4 * s + 3:4 * s + 4])
        a_emp.append((1.0 - z0) * n0)

    whh_v = [whhT[s][:] for s in range(3)]
    bhn_v = [bhn[s][:] for s in range(3)]

    def step(t, carry):
        hs = list(carry[0:3])
        aa = list(carry[3:6])
        tt = tp_ref[pl.ds(t, 1), :] * al
        mcol = ((tt >= s_col) & (tt <= e_col))
        ga = gia_ref[pl.ds(t, 1), :]
        for s in range(3):
            r0, r1 = _ROW0[s], _ROW0[s] + _NPAD[s]
            m = mcol[r0:r1]
            grow = gi_ref[s][pl.ds(t, 1), :]
            gh = jnp.dot(hs[s], whh_v[s], preferred_element_type=jnp.float32)
            r = jax.nn.sigmoid(grow[:, 0:128] + gh[:, 0:128])
            z = jax.nn.sigmoid(grow[:, 128:256] + gh[:, 128:256])
            nn = jnp.tanh(grow[:, 256:384] + r * (gh[:, 256:384] + bhn_v[s]))
            hnew = (1.0 - z) * nn + z * hs[s]
            hs[s] = jnp.where(m, hnew, hs[s])
            gr = ga[:, 3 * s:3 * s + 1]
            gz = ga[:, 3 * s + 1:3 * s + 2]
            gn = ga[:, 3 * s + 2:3 * s + 3]
            wr = abhh[:, 4 * s:4 * s + 1]
            wz = abhh[:, 4 * s + 1:4 * s + 2]
            wn = abhh[:, 4 * s + 2:4 * s + 3]
            bn = abhh[:, 4 * s + 3:4 * s + 4]
            ra = jax.nn.sigmoid(gr + wr * aa[s])
            za = jax.nn.sigmoid(gz + wz * aa[s])
            na = jnp.tanh(gn + ra * (wn * aa[s] + bn))
            anew = (1.0 - za) * na + za * aa[s]
            aa[s] = jnp.where(m, anew, aa[s])
        return tuple(hs) + tuple(aa)

    init = tuple(jnp.zeros((_NPAD[s], _D), jnp.float32) for s in range(3)) + \
           tuple(jnp.zeros((_NPAD[s], 1), jnp.float32) for s in range(3))
    fin = jax.lax.fori_loop(0, _NT, step, init)

    for s in range(3):
        r0, r1 = _ROW0[s], _ROW0[s] + _NPAD[s]
        cnt = counts[r0:r1]
        nonempty = cnt > 0.0
        feat = jnp.where(nonempty, fin[s], h_emp[s])
        abf = jnp.where(nonempty, fin[3 + s], a_emp[s])
        s0 = s_col[r0:r1]
        e0 = e_col[r0:r1]
        ca = ((s0 + e0) / 2.0) / al
        wa = (e0 - s0) / al
        scur, ecur = s0, e0
        nb = _NBINS[s]
        wpv = wp[s][:]
        for l in range(3):
            w1T, b1, w2T, b2 = refw[s][l]
            q = jnp.concatenate([feat, ca, wa, scur / al, ecur / al, abf], axis=1)
            hh = jnp.maximum(jnp.dot(q, w1T[:], preferred_element_type=jnp.float32)
                             + b1[:], 0.0)
            lg = jnp.dot(hh, w2T[:], preferred_element_type=jnp.float32) + b2[:]
            sl = lg[:, 0:nb]
            el = lg[:, nb:2 * nb]
            scur = scur + jnp.sum(jax.nn.softmax(sl, axis=1) * wpv,
                                  axis=1, keepdims=True)
            ecur = ecur + jnp.sum(jax.nn.softmax(el, axis=1) * wpv,
                                  axis=1, keepdims=True)
        wcT, bc, wkT, bk = heads[s]
        lff = jnp.concatenate([feat, abf], axis=1)
        conf = jnp.dot(lff, wcT[:], preferred_element_type=jnp.float32) + bc[:]
        cls = jnp.dot(lff, wkT[:], preferred_element_type=jnp.float32) + bk[:]
        packed = jnp.concatenate(
            [scur, ecur, cls, conf, jnp.zeros((_NPAD[s], 1), jnp.float32)], axis=1)
        out_ref[r0:r1, :] = packed


def kernel(node_embeddings, time_positions, node_pred, audio_len,
           cur_anchor_intervals, params, num_intervals_per_scale):
    f32 = jnp.float32
    s70 = cur_anchor_intervals[:, 0]
    e70 = cur_anchor_intervals[:, 1]
    pad_s = jnp.full((2,), 9.0, f32)
    pad_e = jnp.full((2,), -9.0, f32)
    s_col = jnp.concatenate([s70[0:30], pad_s, s70[30:54], s70[54:70]]).reshape(72, 1)
    e_col = jnp.concatenate([e70[0:30], pad_e, e70[30:54], e70[54:70]]).reshape(72, 1)

    ins = [node_embeddings,
           time_positions.reshape(_NT, 1),
           time_positions.reshape(1, _NT),
           audio_len.reshape(1, 1),
           s_col, e_col,
           node_pred]
    for s in range(3):
        wih, whh, bih, bhh = params['gru_feat'][s]
        ins.append(wih.T)
    for s in range(3):
        wih, whh, bih, bhh = params['gru_feat'][s]
        ins.append(whh.T)
    for s in range(3):
        wih, whh, bih, bhh = params['gru_feat'][s]
        ins.append((bih + jnp.concatenate([bhh[0:256], jnp.zeros((128,), f32)]))
                   .reshape(1, 384))
    for s in range(3):
        wih, whh, bih, bhh = params['gru_feat'][s]
        ins.append(bhh[256:384].reshape(1, 128))
    giaw, giab, abhh = [], [], []
    for s in range(3):
        wiha, whha, biha, bhha = params['gru_ab'][s]
        w3 = wiha[:, 0]
        giaw.append(w3)
        giab.append(jnp.stack([biha[0] + bhha[0], biha[1] + bhha[1], biha[2]]))
        abhh.append(jnp.stack([whha[0, 0], whha[1, 0], whha[2, 0], bhha[2]]))
    ins.append(jnp.concatenate(giaw).reshape(1, 9))
    ins.append(jnp.concatenate(giab).reshape(1, 9))
    ins.append(jnp.concatenate(abhh).reshape(1, 12))
    for s in range(3):
        for l in range(3):
            w1, b1, w2, b2 = params['refine'][s][l]
            ins.extend([w1.T, b1.reshape(1, -1), w2.T, b2.reshape(1, -1)])
    for s in range(3):
        ins.append(params['wparams'][s].reshape(1, -1))
    for s in range(3):
        wc, bc = params['conf'][s]
        wk, bk = params['cls'][s]
        ins.extend([wc.T, bc.reshape(1, 1), wk.T, bk.reshape(1, 4)])

    out = pl.pallas_call(
        _body,
        out_shape=jax.ShapeDtypeStruct((72, 8), f32),
        scratch_shapes=[pltpu.VMEM((_NT, 384), f32),
                        pltpu.VMEM((_NT, 384), f32),
                        pltpu.VMEM((_NT, 384), f32),
                        pltpu.VMEM((_NT, 9), f32)],
    )(*ins)

    fb = jnp.concatenate([out[0:30, 0:2], out[32:56, 0:2], out[56:72, 0:2]], axis=0)
    cls = jnp.concatenate([out[0:30, 2:6], out[32:56, 2:6], out[56:72, 2:6]], axis=0)
    conf = jnp.concatenate([out[0:30, 6:7], out[32:56, 6:7], out[56:72, 6:7]], axis=0)
    return fb, cls, conf


# speedup vs baseline: 42.2953x; 1.8683x over previous
import jax
import jax.numpy as jnp
import numpy as np
from jax.experimental import pallas as pl
from jax.experimental.pallas import tpu as pltpu

_SMOOTH = np.array([0.06136, 0.24477, 0.38774, 0.24477, 0.06136], dtype=np.float32)
_NIP = (30, 24, 16)
_NPAD = (32, 24, 16)
_ROW0 = (0, 32, 56)
_NBINS = (80, 60, 40)
_NT = 4096
_D = 128


def _body(*refs):
    it = iter(refs)
    ne_ref = next(it)
    tp_ref = next(it)
    tprow_ref = next(it)
    au_ref = next(it)
    scol_ref = next(it)
    ecol_ref = next(it)
    srow_ref = next(it)
    erow_ref = next(it)
    npp_ref = next(it)
    wihT = [next(it) for _ in range(3)]
    whhT = [next(it) for _ in range(3)]
    gb = [next(it) for _ in range(3)]
    bhn = [next(it) for _ in range(3)]
    abwi_ref = next(it)
    abbi_ref = next(it)
    abhh_ref = next(it)
    refw = [[(next(it), next(it), next(it), next(it)) for _ in range(3)]
            for _ in range(3)]
    wp = [next(it) for _ in range(3)]
    heads = [(next(it), next(it), next(it), next(it)) for _ in range(3)]
    out_ref = next(it)
    gi_ref = [next(it) for _ in range(3)]
    giar_ref = next(it)
    giaz_ref = next(it)
    gian_ref = next(it)

    al = au_ref[:]
    s_col = scol_ref[:]
    e_col = ecol_ref[:]
    s_row = srow_ref[:]
    e_row = erow_ref[:]

    x = npp_ref[:]
    zpad = jnp.zeros((2, 5), jnp.float32)
    xp = jnp.concatenate([zpad, x, zpad], axis=0)
    sm = (_SMOOTH[0] * xp[0:_NT] + _SMOOTH[1] * xp[1:_NT + 1]
          + _SMOOTH[2] * xp[2:_NT + 2] + _SMOOTH[3] * xp[3:_NT + 3]
          + _SMOOTH[4] * xp[4:_NT + 4])
    smax = jax.nn.softmax(sm, axis=1)
    ab_col = 1.0 - smax[:, 0:1]
    abwi = abwi_ref[:]
    abbi = abbi_ref[:]
    giar_ref[:] = ab_col * abwi[0:1, :] + abbi[0:1, :]
    giaz_ref[:] = ab_col * abwi[1:2, :] + abbi[1:2, :]
    gian_ref[:] = ab_col * abwi[2:3, :] + abbi[2:3, :]
    abhh = abhh_ref[:]
    wr72 = abhh[0:1, :]
    wz72 = abhh[1:2, :]
    wn72 = abhh[2:3, :]
    bn72 = abhh[3:4, :]

    nev = ne_ref[:]
    for s in range(3):
        gi_ref[s][:] = jnp.dot(nev, wihT[s][:],
                               preferred_element_type=jnp.float32) + gb[s][:]

    t_row = tprow_ref[:] * al
    inmask = (t_row >= s_col) & (t_row <= e_col)
    counts = jnp.sum(inmask.astype(jnp.float32), axis=1, keepdims=True)

    h_emp = []
    for s in range(3):
        gbv = gb[s][:]
        r0 = jax.nn.sigmoid(gbv[:, 0:128])
        z0 = jax.nn.sigmoid(gbv[:, 128:256])
        n0 = jnp.tanh(gbv[:, 256:384] + r0 * bhn[s][:])
        h_emp.append((1.0 - z0) * n0)
    ra0 = jax.nn.sigmoid(abbi[0:1, :])
    za0 = jax.nn.sigmoid(abbi[1:2, :])
    na0 = jnp.tanh(abbi[2:3, :] + ra0 * bn72)
    a_emp_row = (1.0 - za0) * na0

    whh_v = [whhT[s][:] for s in range(3)]
    bhn_v = [bhn[s][:] for s in range(3)]

    def step(t, carry):
        hs = list(carry[0:3])
        a = carry[3]
        tt = tp_ref[pl.ds(t, 1), :] * al
        mcol = (tt >= s_col) & (tt <= e_col)
        mrow = (tt >= s_row) & (tt <= e_row)
        for s in range(3):
            r0, r1 = _ROW0[s], _ROW0[s] + _NPAD[s]
            m = mcol[r0:r1]
            grow = gi_ref[s][pl.ds(t, 1), :]
            gh = jnp.dot(hs[s], whh_v[s], preferred_element_type=jnp.float32)
            r = jax.nn.sigmoid(grow[:, 0:128] + gh[:, 0:128])
            z = jax.nn.sigmoid(grow[:, 128:256] + gh[:, 128:256])
            nn = jnp.tanh(grow[:, 256:384] + r * (gh[:, 256:384] + bhn_v[s]))
            hnew = nn + z * (hs[s] - nn)
            hs[s] = jnp.where(m, hnew, hs[s])
        gr = giar_ref[pl.ds(t, 1), :]
        gz = giaz_ref[pl.ds(t, 1), :]
        gn = gian_ref[pl.ds(t, 1), :]
        ra = jax.nn.sigmoid(gr + wr72 * a)
        za = jax.nn.sigmoid(gz + wz72 * a)
        na = jnp.tanh(gn + ra * (wn72 * a + bn72))
        anew = na + za * (a - na)
        a = jnp.where(mrow, anew, a)
        return tuple(hs) + (a,)

    init = tuple(jnp.zeros((_NPAD[s], _D), jnp.float32) for s in range(3)) + \
           (jnp.zeros((1, 72), jnp.float32),)
    fin = jax.lax.fori_loop(0, _NT, step, init)

    afin_col = jnp.swapaxes(fin[3], 0, 1)
    for s in range(3):
        r0, r1 = _ROW0[s], _ROW0[s] + _NPAD[s]
        cnt = counts[r0:r1]
        nonempty = cnt > 0.0
        feat = jnp.where(nonempty, fin[s], h_emp[s])
        a_emp_col = jnp.swapaxes(a_emp_row[:, r0:r1], 0, 1)
        abf = jnp.where(nonempty, afin_col[r0:r1], a_emp_col)
        s0 = s_col[r0:r1]
        e0 = e_col[r0:r1]
        ca = ((s0 + e0) / 2.0) / al
        wa = (e0 - s0) / al
        scur, ecur = s0, e0
        nb = _NBINS[s]
        wpv = wp[s][:]
        for l in range(3):
            w1T, b1, w2T, b2 = refw[s][l]
            q = jnp.concatenate([feat, ca, wa, scur / al, ecur / al, abf], axis=1)
            hh = jnp.maximum(jnp.dot(q, w1T[:], preferred_element_type=jnp.float32)
                             + b1[:], 0.0)
            lg = jnp.dot(hh, w2T[:], preferred_element_type=jnp.float32) + b2[:]
            sl = lg[:, 0:nb]
            el = lg[:, nb:2 * nb]
            scur = scur + jnp.sum(jax.nn.softmax(sl, axis=1) * wpv,
                                  axis=1, keepdims=True)
            ecur = ecur + jnp.sum(jax.nn.softmax(el, axis=1) * wpv,
                                  axis=1, keepdims=True)
        wcT, bc, wkT, bk = heads[s]
        lff = jnp.concatenate([feat, abf], axis=1)
        conf = jnp.dot(lff, wcT[:], preferred_element_type=jnp.float32) + bc[:]
        cls = jnp.dot(lff, wkT[:], preferred_element_type=jnp.float32) + bk[:]
        packed = jnp.concatenate(
            [scur, ecur, cls, conf, jnp.zeros((_NPAD[s], 1), jnp.float32)], axis=1)
        out_ref[r0:r1, :] = packed


def _rep72(vals):
    return jnp.concatenate([jnp.full((_NPAD[s],), vals[s], jnp.float32)
                            for s in range(3)]).reshape(1, 72)


def kernel(node_embeddings, time_positions, node_pred, audio_len,
           cur_anchor_intervals, params, num_intervals_per_scale):
    f32 = jnp.float32
    s70 = cur_anchor_intervals[:, 0]
    e70 = cur_anchor_intervals[:, 1]
    pad_s = jnp.full((2,), 9.0, f32)
    pad_e = jnp.full((2,), -9.0, f32)
    s_flat = jnp.concatenate([s70[0:30], pad_s, s70[30:54], s70[54:70]])
    e_flat = jnp.concatenate([e70[0:30], pad_e, e70[30:54], e70[54:70]])

    ins = [node_embeddings,
           time_positions.reshape(_NT, 1),
           time_positions.reshape(1, _NT),
           audio_len.reshape(1, 1),
           s_flat.reshape(72, 1), e_flat.reshape(72, 1),
           s_flat.reshape(1, 72), e_flat.reshape(1, 72),
           node_pred]
    for s in range(3):
        wih, whh, bih, bhh = params['gru_feat'][s]
        ins.append(wih.T)
    for s in range(3):
        wih, whh, bih, bhh = params['gru_feat'][s]
        ins.append(whh.T)
    for s in range(3):
        wih, whh, bih, bhh = params['gru_feat'][s]
        ins.append((bih + jnp.concatenate([bhh[0:256], jnp.zeros((128,), f32)]))
                   .reshape(1, 384))
    for s in range(3):
        wih, whh, bih, bhh = params['gru_feat'][s]
        ins.append(bhh[256:384].reshape(1, 128))
    abp = [params['gru_ab'][s] for s in range(3)]
    abwi = jnp.concatenate([_rep72([abp[s][0][g, 0] for s in range(3)])
                            for g in range(3)], axis=0)
    abbi = jnp.concatenate(
        [_rep72([abp[s][2][0] + abp[s][3][0] for s in range(3)]),
         _rep72([abp[s][2][1] + abp[s][3][1] for s in range(3)]),
         _rep72([abp[s][2][2] for s in range(3)])], axis=0)
    abhh = jnp.concatenate(
        [_rep72([abp[s][1][0, 0] for s in range(3)]),
         _rep72([abp[s][1][1, 0] for s in range(3)]),
         _rep72([abp[s][1][2, 0] for s in range(3)]),
         _rep72([abp[s][3][2] for s in range(3)])], axis=0)
    ins.extend([abwi, abbi, abhh])
    for s in range(3):
        for l in range(3):
            w1, b1, w2, b2 = params['refine'][s][l]
            ins.extend([w1.T, b1.reshape(1, -1), w2.T, b2.reshape(1, -1)])
    for s in range(3):
        ins.append(params['wparams'][s].reshape(1, -1))
    for s in range(3):
        wc, bc = params['conf'][s]
        wk, bk = params['cls'][s]
        ins.extend([wc.T, bc.reshape(1, 1), wk.T, bk.reshape(1, 4)])

    out = pl.pallas_call(
        _body,
        out_shape=jax.ShapeDtypeStruct((72, 8), f32),
        scratch_shapes=[pltpu.VMEM((_NT, 384), f32),
                        pltpu.VMEM((_NT, 384), f32),
                        pltpu.VMEM((_NT, 384), f32),
                        pltpu.VMEM((_NT, 72), f32),
                        pltpu.VMEM((_NT, 72), f32),
                        pltpu.VMEM((_NT, 72), f32)],
    )(*ins)

    fb = jnp.concatenate([out[0:30, 0:2], out[32:56, 0:2], out[56:72, 0:2]], axis=0)
    cls = jnp.concatenate([out[0:30, 2:6], out[32:56, 2:6], out[56:72, 2:6]], axis=0)
    conf = jnp.concatenate([out[0:30, 6:7], out[32:56, 6:7], out[56:72, 6:7]], axis=0)
    return fb, cls, conf
